# Initial kernel scaffold; baseline (speedup 1.0000x reference)
import jax
import jax.numpy as jnp
from jax.experimental import pallas as pl


def kernel(x, edge_index, edge_attr, batch, atom_emb, bond_emb, eps, W1, b1, W2, b2, Wc, bc):
    raise NotImplementedError("write your pallas kernel here")


# speedup vs baseline: 5.0097x; 5.0097x over previous
import functools

import jax
import jax.numpy as jnp
from jax import lax
from jax.experimental import pallas as pl
from jax.experimental.pallas import tpu as pltpu
from jax.experimental.pallas import tpu_sc as plsc

N_NODES = 10000
N_EDGES = 320000
HIDDEN = 128
NUM_LAYERS = 3
NUM_GRAPHS = 128
OUT_DIM = 10
N_ATOM_FEATS = 9
N_BOND_FEATS = 3
ATOM_VOCAB = 128
BOND_VOCAB = 16

NC = 2
NS = 16
NW = NC * NS
CHUNK = 128
CPW = 79
E_PAD = NW * CPW * CHUNK
N_PAD = 10240
ROWS_PER_TILE = N_PAD // NS
VEC = 16


def _atom_encoder_body(x_ref, emb_ref, out_ref):
    xb = x_ref[...]
    blk = xb.shape[0]
    acc = jnp.zeros((blk, HIDDEN), dtype=jnp.float32)
    iota = lax.broadcasted_iota(jnp.int32, (blk, ATOM_VOCAB), 1)
    for f in range(N_ATOM_FEATS):
        onehot = (xb[:, f][:, None] == iota).astype(jnp.float32)
        acc = acc + jnp.dot(onehot, emb_ref[f],
                            preferred_element_type=jnp.float32)
    out_ref[...] = acc


def _atom_encoder(x, atom_emb):
    blk = 1000
    grid = N_NODES // blk
    return pl.pallas_call(
        _atom_encoder_body,
        grid=(grid,),
        in_specs=[
            pl.BlockSpec((blk, N_ATOM_FEATS), lambda i: (i, 0)),
            pl.BlockSpec((N_ATOM_FEATS, ATOM_VOCAB, HIDDEN), lambda i: (0, 0, 0)),
        ],
        out_specs=pl.BlockSpec((blk, HIDDEN), lambda i: (i, 0)),
        out_shape=jax.ShapeDtypeStruct((N_NODES, HIDDEN), jnp.float32),
    )(x, atom_emb)


def _bond_tables_body(bond_ref, out_ref):
    for l in range(NUM_LAYERS):
        t0 = bond_ref[l, 0]
        t1 = bond_ref[l, 1]
        t2 = bond_ref[l, 2]
        t = (t2[:, None, None, :] + t1[None, :, None, :] + t0[None, None, :, :])
        out_ref[l] = t.reshape(BOND_VOCAB ** 3, HIDDEN)


def _bond_tables(bond_emb):
    return pl.pallas_call(
        _bond_tables_body,
        out_shape=jax.ShapeDtypeStruct(
            (NUM_LAYERS, BOND_VOCAB ** 3, HIDDEN), jnp.float32),
    )(bond_emb)


def _sc_edge_body(h_hbm, src_hbm, dst_hbm, eidx_hbm, table_hbm,
                  out0_hbm, out1_hbm,
                  src_v, dst_v, eidx_v, hrows_v, erows_v,
                  aggr_sh, sem_h, sem_e):
    c = lax.axis_index("c")
    s = lax.axis_index("s")
    w = c * NS + s

    def _zfill(j, _):
        for k in range(HIDDEN // VEC):
            hrows_v[j, pl.ds(k * VEC, VEC)] = jnp.zeros((VEC,), jnp.float32)
        return 0
    lax.fori_loop(0, CHUNK, _zfill, 0)
    for r in range(ROWS_PER_TILE // CHUNK):
        pltpu.sync_copy(
            hrows_v, aggr_sh.at[pl.ds(s * ROWS_PER_TILE + r * CHUNK, CHUNK)])
    plsc.subcore_barrier()

    def chunk_body(ci, _):
        base = (w * CPW + ci) * CHUNK
        pltpu.sync_copy(src_hbm.at[pl.ds(base, CHUNK)], src_v)
        pltpu.sync_copy(eidx_hbm.at[pl.ds(base, CHUNK)], eidx_v)
        pltpu.sync_copy(dst_hbm.at[pl.ds(base, CHUNK)], dst_v)
        cp_h = pltpu.async_copy(h_hbm.at[src_v], hrows_v, sem_h)
        cp_e = pltpu.async_copy(table_hbm.at[eidx_v], erows_v, sem_e)
        cp_h.wait()
        cp_e.wait()

        def relu_body(j, _):
            for k in range(HIDDEN // VEC):
                sl = pl.ds(k * VEC, VEC)
                hrows_v[j, sl] = jnp.maximum(hrows_v[j, sl] + erows_v[j, sl], 0.0)
            return 0
        lax.fori_loop(0, CHUNK, relu_body, 0)

        pltpu.sync_copy(hrows_v, aggr_sh.at[dst_v], add=True)
        return 0
    lax.fori_loop(0, CPW, chunk_body, 0)
    plsc.subcore_barrier()

    @pl.when(c == 0)
    def _():
        pltpu.sync_copy(aggr_sh.at[pl.ds(s * ROWS_PER_TILE, ROWS_PER_TILE)],
                        out0_hbm.at[pl.ds(s * ROWS_PER_TILE, ROWS_PER_TILE)])

    @pl.when(c == 1)
    def _():
        pltpu.sync_copy(aggr_sh.at[pl.ds(s * ROWS_PER_TILE, ROWS_PER_TILE)],
                        out1_hbm.at[pl.ds(s * ROWS_PER_TILE, ROWS_PER_TILE)])


@functools.cache
def _get_sc_edge():
  return pl.kernel(
    _sc_edge_body,
    out_type=(
        jax.ShapeDtypeStruct((N_PAD, HIDDEN), jnp.float32),
        jax.ShapeDtypeStruct((N_PAD, HIDDEN), jnp.float32),
    ),
    mesh=plsc.VectorSubcoreMesh(core_axis_name="c", subcore_axis_name="s",
                                num_cores=NC, num_subcores=NS),
    scratch_types=[
        pltpu.VMEM((CHUNK,), jnp.int32),
        pltpu.VMEM((CHUNK,), jnp.int32),
        pltpu.VMEM((CHUNK,), jnp.int32),
        pltpu.VMEM((CHUNK, HIDDEN), jnp.float32),
        pltpu.VMEM((CHUNK, HIDDEN), jnp.float32),
        pltpu.VMEM_SHARED((N_PAD, HIDDEN), jnp.float32),
        pltpu.SemaphoreType.DMA,
        pltpu.SemaphoreType.DMA,
    ],
  )


def _mlp_body(h_ref, a0_ref, a1_ref, scale_ref, w1_ref, b1_ref, w2_ref, b2_ref,
              out_ref):
    z = h_ref[...] * scale_ref[0, 0] + a0_ref[...] + a1_ref[...]
    t = jnp.dot(z, w1_ref[...], preferred_element_type=jnp.float32) + b1_ref[...]
    t = jnp.maximum(t, 0.0)
    out_ref[...] = (jnp.dot(t, w2_ref[...], preferred_element_type=jnp.float32)
                    + b2_ref[...])


def _mlp(h, a0, a1, scale, w1, b1, w2, b2):
    blk = 1000
    grid = N_NODES // blk
    return pl.pallas_call(
        _mlp_body,
        grid=(grid,),
        in_specs=[
            pl.BlockSpec((blk, HIDDEN), lambda i: (i, 0)),
            pl.BlockSpec((blk, HIDDEN), lambda i: (i, 0)),
            pl.BlockSpec((blk, HIDDEN), lambda i: (i, 0)),
            pl.BlockSpec((1, 1), lambda i: (0, 0)),
            pl.BlockSpec((HIDDEN, 2 * HIDDEN), lambda i: (0, 0)),
            pl.BlockSpec((1, 2 * HIDDEN), lambda i: (0, 0)),
            pl.BlockSpec((2 * HIDDEN, HIDDEN), lambda i: (0, 0)),
            pl.BlockSpec((1, HIDDEN), lambda i: (0, 0)),
        ],
        out_specs=pl.BlockSpec((blk, HIDDEN), lambda i: (i, 0)),
        out_shape=jax.ShapeDtypeStruct((N_NODES, HIDDEN), jnp.float32),
    )(h, a0, a1, scale, w1, b1, w2, b2)


def _readout_body(h_ref, batch_ref, wc_ref, bc_ref, logits_ref, gf_ref):
    onehot = (batch_ref[...] ==
              lax.broadcasted_iota(jnp.int32, (N_NODES, NUM_GRAPHS), 1)
              ).astype(jnp.float32)
    sums = lax.dot_general(onehot, h_ref[...], (((0,), (0,)), ((), ())),
                           preferred_element_type=jnp.float32)
    counts = jnp.sum(onehot, axis=0)[:, None]
    gf = sums / jnp.maximum(counts, 1.0)
    logits_ref[...] = (jnp.dot(gf, wc_ref[...], preferred_element_type=jnp.float32)
                       + bc_ref[...])
    gf_ref[...] = gf


def _readout(h, batch2d, wc, bc):
    return pl.pallas_call(
        _readout_body,
        out_shape=(
            jax.ShapeDtypeStruct((NUM_GRAPHS, OUT_DIM), jnp.float32),
            jax.ShapeDtypeStruct((NUM_GRAPHS, HIDDEN), jnp.float32),
        ),
    )(h, batch2d, wc, bc)


def kernel(x, edge_index, edge_attr, batch, atom_emb, bond_emb, eps,
           W1, b1, W2, b2, Wc, bc):
    src = edge_index[0]
    dst = edge_index[1]
    eidx = (edge_attr[:, 0] + BOND_VOCAB * edge_attr[:, 1]
            + BOND_VOCAB * BOND_VOCAB * edge_attr[:, 2])
    pad = E_PAD - N_EDGES
    src_p = jnp.concatenate([src, jnp.zeros((pad,), jnp.int32)])
    dst_p = jnp.concatenate([dst, jnp.full((pad,), N_NODES, jnp.int32)])
    eidx_p = jnp.concatenate([eidx, jnp.zeros((pad,), jnp.int32)])

    tables = _bond_tables(bond_emb)
    h = _atom_encoder(x, atom_emb)

    b1_2d = b1.reshape(NUM_LAYERS, 1, 2 * HIDDEN)
    b2_2d = b2.reshape(NUM_LAYERS, 1, HIDDEN)
    scales = (1.0 + eps).reshape(NUM_LAYERS, 1, 1)

    for l in range(NUM_LAYERS):
        a0, a1 = _get_sc_edge()(h, src_p, dst_p, eidx_p, tables[l])
        h = _mlp(h, a0, a1, scales[l], W1[l], b1_2d[l], W2[l], b2_2d[l])

    logits, gf = _readout(h, batch[:, None], Wc, bc)
    return (logits, gf)


# speedup vs baseline: 8.0383x; 1.6046x over previous
import functools

import jax
import jax.numpy as jnp
from jax import lax
from jax.experimental import pallas as pl
from jax.experimental.pallas import tpu as pltpu
from jax.experimental.pallas import tpu_sc as plsc

N_NODES = 10000
N_EDGES = 320000
HIDDEN = 128
NUM_LAYERS = 3
NUM_GRAPHS = 128
OUT_DIM = 10
N_ATOM_FEATS = 9
N_BOND_FEATS = 3
ATOM_VOCAB = 128
BOND_VOCAB = 16

NC = 2
NS = 16
NW = NC * NS
CHUNK = 80
CPW = 126
E_PAD = NW * CPW * CHUNK
N_PAD = 10240
ROWS_PER_TILE = N_PAD // NS
VEC = 16


def _atom_encoder_body(x_ref, emb_ref, out_ref):
    xb = x_ref[...]
    blk = xb.shape[0]
    acc = jnp.zeros((blk, HIDDEN), dtype=jnp.float32)
    iota = lax.broadcasted_iota(jnp.int32, (blk, ATOM_VOCAB), 1)
    for f in range(N_ATOM_FEATS):
        onehot = (xb[:, f][:, None] == iota).astype(jnp.float32)
        acc = acc + jnp.dot(onehot, emb_ref[f],
                            preferred_element_type=jnp.float32)
    out_ref[...] = acc


def _atom_encoder(x, atom_emb):
    blk = 1000
    grid = N_NODES // blk
    return pl.pallas_call(
        _atom_encoder_body,
        grid=(grid,),
        in_specs=[
            pl.BlockSpec((blk, N_ATOM_FEATS), lambda i: (i, 0)),
            pl.BlockSpec((N_ATOM_FEATS, ATOM_VOCAB, HIDDEN), lambda i: (0, 0, 0)),
        ],
        out_specs=pl.BlockSpec((blk, HIDDEN), lambda i: (i, 0)),
        out_shape=jax.ShapeDtypeStruct((N_NODES, HIDDEN), jnp.float32),
    )(x, atom_emb)


def _bond_tables_body(bond_ref, out_ref):
    for l in range(NUM_LAYERS):
        t0 = bond_ref[l, 0]
        t1 = bond_ref[l, 1]
        t2 = bond_ref[l, 2]
        t = (t2[:, None, None, :] + t1[None, :, None, :] + t0[None, None, :, :])
        out_ref[l] = t.reshape(BOND_VOCAB ** 3, HIDDEN)


def _bond_tables(bond_emb):
    return pl.pallas_call(
        _bond_tables_body,
        out_shape=jax.ShapeDtypeStruct(
            (NUM_LAYERS, BOND_VOCAB ** 3, HIDDEN), jnp.float32),
    )(bond_emb)


def _sc_edge_body(h_hbm, src_hbm, dst_hbm, eidx_hbm, table_hbm,
                  out0_hbm, out1_hbm,
                  src_v, dst_v, eidx_v, hrows_v, erows_v,
                  sem_i, sem_g, sem_s,
                  aggr_sh):
    c = lax.axis_index("c")
    s = lax.axis_index("s")
    w = c * NS + s

    def _zfill(j, _):
        for k in range(HIDDEN // VEC):
            hrows_v[0, j, pl.ds(k * VEC, VEC)] = jnp.zeros((VEC,), jnp.float32)
        return 0
    lax.fori_loop(0, CHUNK, _zfill, 0)
    for r in range(ROWS_PER_TILE // CHUNK):
        pltpu.sync_copy(
            hrows_v.at[0], aggr_sh.at[pl.ds(s * ROWS_PER_TILE + r * CHUNK, CHUNK)])
    plsc.subcore_barrier()

    def load_idx(b, ci):
        base = (w * CPW + ci) * CHUNK
        c0 = pltpu.async_copy(src_hbm.at[pl.ds(base, CHUNK)], src_v.at[b], sem_i)
        c1 = pltpu.async_copy(eidx_hbm.at[pl.ds(base, CHUNK)], eidx_v.at[b], sem_i)
        c2 = pltpu.async_copy(dst_hbm.at[pl.ds(base, CHUNK)], dst_v.at[b], sem_i)
        c0.wait(); c1.wait(); c2.wait()

    def fire_gather(b):
        pltpu.async_copy(h_hbm.at[src_v.at[b]], hrows_v.at[b], sem_g)
        pltpu.async_copy(table_hbm.at[eidx_v.at[b]], erows_v.at[b], sem_g)

    def wait_gather(b):
        pltpu.make_async_copy(h_hbm.at[src_v.at[b]], hrows_v.at[b], sem_g).wait()
        pltpu.make_async_copy(table_hbm.at[eidx_v.at[b]], erows_v.at[b], sem_g).wait()

    def fire_scatter(b):
        pltpu.async_copy(hrows_v.at[b], aggr_sh.at[dst_v.at[b]], sem_s, add=True)

    def wait_scatter(b):
        pltpu.make_async_copy(hrows_v.at[b], aggr_sh.at[dst_v.at[b]], sem_s).wait()

    def compute(b):
        def relu_body(j, _):
            for k in range(HIDDEN // VEC):
                sl = pl.ds(k * VEC, VEC)
                hrows_v[b, j, sl] = jnp.maximum(
                    hrows_v[b, j, sl] + erows_v[b, j, sl], 0.0)
            return 0
        lax.fori_loop(0, CHUNK, relu_body, 0)

    load_idx(0, 0)
    fire_gather(0)

    def round_body(r, _):
        @pl.when(r >= 1)
        def _():
            wait_scatter(1)
        load_idx(1, 2 * r + 1)
        fire_gather(1)
        wait_gather(0)
        compute(0)
        fire_scatter(0)

        wait_scatter(0)
        @pl.when(r <= (CPW // 2) - 2)
        def _():
            load_idx(0, 2 * r + 2)
            fire_gather(0)
        wait_gather(1)
        compute(1)
        fire_scatter(1)
        return 0
    lax.fori_loop(0, CPW // 2, round_body, 0)
    wait_scatter(1)
    plsc.subcore_barrier()

    @pl.when(c == 0)
    def _():
        pltpu.sync_copy(aggr_sh.at[pl.ds(s * ROWS_PER_TILE, ROWS_PER_TILE)],
                        out0_hbm.at[pl.ds(s * ROWS_PER_TILE, ROWS_PER_TILE)])

    @pl.when(c == 1)
    def _():
        pltpu.sync_copy(aggr_sh.at[pl.ds(s * ROWS_PER_TILE, ROWS_PER_TILE)],
                        out1_hbm.at[pl.ds(s * ROWS_PER_TILE, ROWS_PER_TILE)])


@functools.cache
def _get_sc_edge():
  return pl.kernel(
    _sc_edge_body,
    out_type=(
        jax.ShapeDtypeStruct((N_PAD, HIDDEN), jnp.float32),
        jax.ShapeDtypeStruct((N_PAD, HIDDEN), jnp.float32),
    ),
    mesh=plsc.VectorSubcoreMesh(core_axis_name="c", subcore_axis_name="s",
                                num_cores=NC, num_subcores=NS),
    scratch_types=[
        pltpu.VMEM((2, CHUNK), jnp.int32),
        pltpu.VMEM((2, CHUNK), jnp.int32),
        pltpu.VMEM((2, CHUNK), jnp.int32),
        pltpu.VMEM((2, CHUNK, HIDDEN), jnp.float32),
        pltpu.VMEM((2, CHUNK, HIDDEN), jnp.float32),
        pltpu.SemaphoreType.DMA,
        pltpu.SemaphoreType.DMA,
        pltpu.SemaphoreType.DMA,
        pltpu.VMEM_SHARED((N_PAD, HIDDEN), jnp.float32),
    ],
  )


def _mlp_body(h_ref, a0_ref, a1_ref, scale_ref, w1_ref, b1_ref, w2_ref, b2_ref,
              out_ref):
    z = h_ref[...] * scale_ref[0, 0] + a0_ref[...] + a1_ref[...]
    t = jnp.dot(z, w1_ref[...], preferred_element_type=jnp.float32) + b1_ref[...]
    t = jnp.maximum(t, 0.0)
    out_ref[...] = (jnp.dot(t, w2_ref[...], preferred_element_type=jnp.float32)
                    + b2_ref[...])


def _mlp(h, a0, a1, scale, w1, b1, w2, b2):
    blk = 1000
    grid = N_NODES // blk
    return pl.pallas_call(
        _mlp_body,
        grid=(grid,),
        in_specs=[
            pl.BlockSpec((blk, HIDDEN), lambda i: (i, 0)),
            pl.BlockSpec((blk, HIDDEN), lambda i: (i, 0)),
            pl.BlockSpec((blk, HIDDEN), lambda i: (i, 0)),
            pl.BlockSpec((1, 1), lambda i: (0, 0)),
            pl.BlockSpec((HIDDEN, 2 * HIDDEN), lambda i: (0, 0)),
            pl.BlockSpec((1, 2 * HIDDEN), lambda i: (0, 0)),
            pl.BlockSpec((2 * HIDDEN, HIDDEN), lambda i: (0, 0)),
            pl.BlockSpec((1, HIDDEN), lambda i: (0, 0)),
        ],
        out_specs=pl.BlockSpec((blk, HIDDEN), lambda i: (i, 0)),
        out_shape=jax.ShapeDtypeStruct((N_NODES, HIDDEN), jnp.float32),
    )(h, a0, a1, scale, w1, b1, w2, b2)


def _readout_body(h_ref, batch_ref, wc_ref, bc_ref, logits_ref, gf_ref):
    onehot = (batch_ref[...] ==
              lax.broadcasted_iota(jnp.int32, (N_NODES, NUM_GRAPHS), 1)
              ).astype(jnp.float32)
    sums = lax.dot_general(onehot, h_ref[...], (((0,), (0,)), ((), ())),
                           preferred_element_type=jnp.float32)
    counts = jnp.sum(onehot, axis=0)[:, None]
    gf = sums / jnp.maximum(counts, 1.0)
    logits_ref[...] = (jnp.dot(gf, wc_ref[...], preferred_element_type=jnp.float32)
                       + bc_ref[...])
    gf_ref[...] = gf


def _readout(h, batch2d, wc, bc):
    return pl.pallas_call(
        _readout_body,
        out_shape=(
            jax.ShapeDtypeStruct((NUM_GRAPHS, OUT_DIM), jnp.float32),
            jax.ShapeDtypeStruct((NUM_GRAPHS, HIDDEN), jnp.float32),
        ),
    )(h, batch2d, wc, bc)


def kernel(x, edge_index, edge_attr, batch, atom_emb, bond_emb, eps,
           W1, b1, W2, b2, Wc, bc):
    src = edge_index[0]
    dst = edge_index[1]
    eidx = (edge_attr[:, 0] + BOND_VOCAB * edge_attr[:, 1]
            + BOND_VOCAB * BOND_VOCAB * edge_attr[:, 2])
    pad = E_PAD - N_EDGES
    src_p = jnp.concatenate([src, jnp.zeros((pad,), jnp.int32)])
    dst_p = jnp.concatenate([dst, jnp.full((pad,), N_NODES, jnp.int32)])
    eidx_p = jnp.concatenate([eidx, jnp.zeros((pad,), jnp.int32)])

    tables = _bond_tables(bond_emb)
    h = _atom_encoder(x, atom_emb)

    b1_2d = b1.reshape(NUM_LAYERS, 1, 2 * HIDDEN)
    b2_2d = b2.reshape(NUM_LAYERS, 1, HIDDEN)
    scales = (1.0 + eps).reshape(NUM_LAYERS, 1, 1)

    for l in range(NUM_LAYERS):
        a0, a1 = _get_sc_edge()(h, src_p, dst_p, eidx_p, tables[l])
        h = _mlp(h, a0, a1, scales[l], W1[l], b1_2d[l], W2[l], b2_2d[l])

    logits, gf = _readout(h, batch[:, None], Wc, bc)
    return (logits, gf)


# speedup vs baseline: 8.0395x; 1.0001x over previous
import functools

import jax
import jax.numpy as jnp
from jax import lax
from jax.experimental import pallas as pl
from jax.experimental.pallas import tpu as pltpu
from jax.experimental.pallas import tpu_sc as plsc

N_NODES = 10000
N_EDGES = 320000
HIDDEN = 128
NUM_LAYERS = 3
NUM_GRAPHS = 128
OUT_DIM = 10
N_ATOM_FEATS = 9
N_BOND_FEATS = 3
ATOM_VOCAB = 128
BOND_VOCAB = 16

NC = 2
NS = 16
NW = NC * NS
CHUNK = 80
CPW = 126
E_PAD = NW * CPW * CHUNK
N_PAD = 10240
ROWS_PER_TILE = N_PAD // NS
VEC = 16


def _atom_encoder_body(x_ref, emb_ref, out_ref):
    xb = x_ref[...]
    blk = xb.shape[0]
    acc = jnp.zeros((blk, HIDDEN), dtype=jnp.float32)
    iota = lax.broadcasted_iota(jnp.int32, (blk, ATOM_VOCAB), 1)
    for f in range(N_ATOM_FEATS):
        onehot = (xb[:, f][:, None] == iota).astype(jnp.float32)
        acc = acc + jnp.dot(onehot, emb_ref[f],
                            preferred_element_type=jnp.float32)
    out_ref[...] = acc


def _atom_encoder(x, atom_emb):
    blk = 1000
    grid = N_NODES // blk
    return pl.pallas_call(
        _atom_encoder_body,
        grid=(grid,),
        in_specs=[
            pl.BlockSpec((blk, N_ATOM_FEATS), lambda i: (i, 0)),
            pl.BlockSpec((N_ATOM_FEATS, ATOM_VOCAB, HIDDEN), lambda i: (0, 0, 0)),
        ],
        out_specs=pl.BlockSpec((blk, HIDDEN), lambda i: (i, 0)),
        out_shape=jax.ShapeDtypeStruct((N_NODES, HIDDEN), jnp.float32),
    )(x, atom_emb)


def _bond_tables_body(bond_ref, out_ref):
    for l in range(NUM_LAYERS):
        t0 = bond_ref[l, 0]
        t1 = bond_ref[l, 1]
        t2 = bond_ref[l, 2]
        t = (t2[:, None, None, :] + t1[None, :, None, :] + t0[None, None, :, :])
        out_ref[l] = t.reshape(BOND_VOCAB ** 3, HIDDEN)


def _bond_tables(bond_emb):
    return pl.pallas_call(
        _bond_tables_body,
        out_shape=jax.ShapeDtypeStruct(
            (NUM_LAYERS, BOND_VOCAB ** 3, HIDDEN), jnp.float32),
    )(bond_emb)


def _sc_edge_body(h_hbm, src_hbm, dst_hbm, eidx_hbm, table_hbm,
                  out0_hbm, out1_hbm,
                  src_v, dst_v, eidx_v, hrows_v, erows_v,
                  sem_i, sem_g, sem_s,
                  aggr_sh):
    c = lax.axis_index("c")
    s = lax.axis_index("s")
    w = c * NS + s

    @plsc.parallel_loop(0, CHUNK, 1, unroll=4)
    def _zfill(j):
        for k in range(HIDDEN // VEC):
            hrows_v[0, j, pl.ds(k * VEC, VEC)] = jnp.zeros((VEC,), jnp.float32)
    for r in range(ROWS_PER_TILE // CHUNK):
        pltpu.sync_copy(
            hrows_v.at[0], aggr_sh.at[pl.ds(s * ROWS_PER_TILE + r * CHUNK, CHUNK)])
    plsc.subcore_barrier()

    def load_idx(b, ci):
        base = (w * CPW + ci) * CHUNK
        c0 = pltpu.async_copy(src_hbm.at[pl.ds(base, CHUNK)], src_v.at[b], sem_i)
        c1 = pltpu.async_copy(eidx_hbm.at[pl.ds(base, CHUNK)], eidx_v.at[b], sem_i)
        c2 = pltpu.async_copy(dst_hbm.at[pl.ds(base, CHUNK)], dst_v.at[b], sem_i)
        c0.wait(); c1.wait(); c2.wait()

    def fire_gather(b):
        pltpu.async_copy(h_hbm.at[src_v.at[b]], hrows_v.at[b], sem_g)
        pltpu.async_copy(table_hbm.at[eidx_v.at[b]], erows_v.at[b], sem_g)

    def wait_gather(b):
        pltpu.make_async_copy(h_hbm.at[src_v.at[b]], hrows_v.at[b], sem_g).wait()
        pltpu.make_async_copy(table_hbm.at[eidx_v.at[b]], erows_v.at[b], sem_g).wait()

    def fire_scatter(b):
        pltpu.async_copy(hrows_v.at[b], aggr_sh.at[dst_v.at[b]], sem_s, add=True)

    def wait_scatter(b):
        pltpu.make_async_copy(hrows_v.at[b], aggr_sh.at[dst_v.at[b]], sem_s).wait()

    def compute(b):
        @plsc.parallel_loop(0, CHUNK, 1, unroll=4)
        def _(j):
            for k in range(HIDDEN // VEC):
                sl = pl.ds(k * VEC, VEC)
                hrows_v[b, j, sl] = jnp.maximum(
                    hrows_v[b, j, sl] + erows_v[b, j, sl], 0.0)

    load_idx(0, 0)
    fire_gather(0)

    def round_body(r, _):
        @pl.when(r >= 1)
        def _():
            wait_scatter(1)
        load_idx(1, 2 * r + 1)
        fire_gather(1)
        wait_gather(0)
        compute(0)
        fire_scatter(0)

        wait_scatter(0)
        @pl.when(r <= (CPW // 2) - 2)
        def _():
            load_idx(0, 2 * r + 2)
            fire_gather(0)
        wait_gather(1)
        compute(1)
        fire_scatter(1)
        return 0
    lax.fori_loop(0, CPW // 2, round_body, 0)
    wait_scatter(1)
    plsc.subcore_barrier()

    @pl.when(c == 0)
    def _():
        pltpu.sync_copy(aggr_sh.at[pl.ds(s * ROWS_PER_TILE, ROWS_PER_TILE)],
                        out0_hbm.at[pl.ds(s * ROWS_PER_TILE, ROWS_PER_TILE)])

    @pl.when(c == 1)
    def _():
        pltpu.sync_copy(aggr_sh.at[pl.ds(s * ROWS_PER_TILE, ROWS_PER_TILE)],
                        out1_hbm.at[pl.ds(s * ROWS_PER_TILE, ROWS_PER_TILE)])


@functools.cache
def _get_sc_edge():
  return pl.kernel(
    _sc_edge_body,
    out_type=(
        jax.ShapeDtypeStruct((N_PAD, HIDDEN), jnp.float32),
        jax.ShapeDtypeStruct((N_PAD, HIDDEN), jnp.float32),
    ),
    mesh=plsc.VectorSubcoreMesh(core_axis_name="c", subcore_axis_name="s",
                                num_cores=NC, num_subcores=NS),
    scratch_types=[
        pltpu.VMEM((2, CHUNK), jnp.int32),
        pltpu.VMEM((2, CHUNK), jnp.int32),
        pltpu.VMEM((2, CHUNK), jnp.int32),
        pltpu.VMEM((2, CHUNK, HIDDEN), jnp.float32),
        pltpu.VMEM((2, CHUNK, HIDDEN), jnp.float32),
        pltpu.SemaphoreType.DMA,
        pltpu.SemaphoreType.DMA,
        pltpu.SemaphoreType.DMA,
        pltpu.VMEM_SHARED((N_PAD, HIDDEN), jnp.float32),
    ],
  )


def _mlp_body(h_ref, a0_ref, a1_ref, scale_ref, w1_ref, b1_ref, w2_ref, b2_ref,
              out_ref):
    z = h_ref[...] * scale_ref[0, 0] + a0_ref[...] + a1_ref[...]
    t = jnp.dot(z, w1_ref[...], preferred_element_type=jnp.float32) + b1_ref[...]
    t = jnp.maximum(t, 0.0)
    out_ref[...] = (jnp.dot(t, w2_ref[...], preferred_element_type=jnp.float32)
                    + b2_ref[...])


def _mlp(h, a0, a1, scale, w1, b1, w2, b2):
    blk = 1000
    grid = N_NODES // blk
    return pl.pallas_call(
        _mlp_body,
        grid=(grid,),
        in_specs=[
            pl.BlockSpec((blk, HIDDEN), lambda i: (i, 0)),
            pl.BlockSpec((blk, HIDDEN), lambda i: (i, 0)),
            pl.BlockSpec((blk, HIDDEN), lambda i: (i, 0)),
            pl.BlockSpec((1, 1), lambda i: (0, 0)),
            pl.BlockSpec((HIDDEN, 2 * HIDDEN), lambda i: (0, 0)),
            pl.BlockSpec((1, 2 * HIDDEN), lambda i: (0, 0)),
            pl.BlockSpec((2 * HIDDEN, HIDDEN), lambda i: (0, 0)),
            pl.BlockSpec((1, HIDDEN), lambda i: (0, 0)),
        ],
        out_specs=pl.BlockSpec((blk, HIDDEN), lambda i: (i, 0)),
        out_shape=jax.ShapeDtypeStruct((N_NODES, HIDDEN), jnp.float32),
    )(h, a0, a1, scale, w1, b1, w2, b2)


def _readout_body(h_ref, batch_ref, wc_ref, bc_ref, logits_ref, gf_ref):
    onehot = (batch_ref[...] ==
              lax.broadcasted_iota(jnp.int32, (N_NODES, NUM_GRAPHS), 1)
              ).astype(jnp.float32)
    sums = lax.dot_general(onehot, h_ref[...], (((0,), (0,)), ((), ())),
                           preferred_element_type=jnp.float32)
    counts = jnp.sum(onehot, axis=0)[:, None]
    gf = sums / jnp.maximum(counts, 1.0)
    logits_ref[...] = (jnp.dot(gf, wc_ref[...], preferred_element_type=jnp.float32)
                       + bc_ref[...])
    gf_ref[...] = gf


def _readout(h, batch2d, wc, bc):
    return pl.pallas_call(
        _readout_body,
        out_shape=(
            jax.ShapeDtypeStruct((NUM_GRAPHS, OUT_DIM), jnp.float32),
            jax.ShapeDtypeStruct((NUM_GRAPHS, HIDDEN), jnp.float32),
        ),
    )(h, batch2d, wc, bc)


def kernel(x, edge_index, edge_attr, batch, atom_emb, bond_emb, eps,
           W1, b1, W2, b2, Wc, bc):
    src = edge_index[0]
    dst = edge_index[1]
    eidx = (edge_attr[:, 0] + BOND_VOCAB * edge_attr[:, 1]
            + BOND_VOCAB * BOND_VOCAB * edge_attr[:, 2])
    pad = E_PAD - N_EDGES
    src_p = jnp.concatenate([src, jnp.zeros((pad,), jnp.int32)])
    dst_p = jnp.concatenate([dst, jnp.full((pad,), N_NODES, jnp.int32)])
    eidx_p = jnp.concatenate([eidx, jnp.zeros((pad,), jnp.int32)])

    tables = _bond_tables(bond_emb)
    h = _atom_encoder(x, atom_emb)

    b1_2d = b1.reshape(NUM_LAYERS, 1, 2 * HIDDEN)
    b2_2d = b2.reshape(NUM_LAYERS, 1, HIDDEN)
    scales = (1.0 + eps).reshape(NUM_LAYERS, 1, 1)

    for l in range(NUM_LAYERS):
        a0, a1 = _get_sc_edge()(h, src_p, dst_p, eidx_p, tables[l])
        h = _mlp(h, a0, a1, scales[l], W1[l], b1_2d[l], W2[l], b2_2d[l])

    logits, gf = _readout(h, batch[:, None], Wc, bc)
    return (logits, gf)
